# baseline (speedup 1.0000x reference)
import functools

import jax
import jax.numpy as jnp
from jax import lax
from jax.experimental import pallas as pl
from jax.experimental.pallas import tpu as pltpu
from jax.experimental.pallas import tpu_sc as plsc

D = 64
L = 16
NC = 2
NS = 16
NW = NC * NS
BT = 128
DT = D // 8
NBUF = 2
DPACK = 1024


def _build(batch, seq, diac_vocab):
    assert batch == NW * BT
    assert seq % 8 == 0
    st = seq // 8
    assert seq % NBUF == 0 and seq >= 2 * NBUF

    mesh = plsc.VectorSubcoreMesh(core_axis_name="c", subcore_axis_name="s")

    @functools.partial(
        pl.kernel,
        out_type=jax.ShapeDtypeStruct((seq, DT, NW, 8, BT), jnp.float32),
        mesh=mesh,
        scratch_types=[
            pltpu.VMEM((st, 1, 8, BT), jnp.int32),
            pltpu.VMEM((NBUF, BT), jnp.int32),
            pltpu.VMEM((diac_vocab, D), jnp.float32),
            pltpu.VMEM((NBUF, BT, D), jnp.float32),
            pltpu.VMEM((NBUF, DT, 1, 8, BT), jnp.float32),
            pltpu.SemaphoreType.DMA((NBUF,)),
            pltpu.SemaphoreType.DMA((NBUF,)),
        ],
        compiler_params=pltpu.CompilerParams(
            use_tc_tiling_on_sc=False, needs_layout_passes=False),
    )
    def kern(pk_idx_hbm, tok_tab_hbm, diac_tab_hbm, out_hbm,
             pk_v, ti_v, dtab_v, tr_v, ob_v, gsems, wsems):
        wid = lax.axis_index("s") * NC + lax.axis_index("c")

        pltpu.sync_copy(pk_idx_hbm.at[:, pl.ds(wid, 1)], pk_v)
        pltpu.sync_copy(diac_tab_hbm, dtab_v)

        iota = lax.iota(jnp.int32, L)
        zvec = jnp.zeros((L,), jnp.int32)
        cvecs = [iota + L * cb for cb in range(BT // L)]

        def pk_row(s):
            return pk_v.at[s // 8, 0, lax.rem(s, 8)]

        def unpack_tok(s, b):
            row = pk_row(s)
            for cb in range(BT // L):
                sl = pl.ds(cb * L, L)
                ti_v[b, sl] = lax.shift_right_logical(row[sl], 10)

        def gather_desc(b):
            return pltpu.make_async_copy(
                tok_tab_hbm.at[ti_v.at[b]], tr_v.at[b], gsems.at[b])

        def write_desc(s, b):
            return pltpu.make_async_copy(
                ob_v.at[b], out_hbm.at[s, pl.ds(0, DT), pl.ds(wid, 1)],
                wsems.at[b])

        def add_group(s, b):
            row = pk_row(s)
            dvecs = [row[pl.ds(cb * L, L)] & (DPACK - 1)
                     for cb in range(BT // L)]

            @plsc.parallel_loop(0, L, unroll=4)
            def _(j):
                pj = lax.rem(iota + j, jnp.int32(L))
                for db in range(D // L):
                    dvec = pj + L * db
                    dtv = dvec // 8
                    drv = lax.rem(dvec, 8)
                    for cb in range(BT // L):
                        acc = (plsc.load_gather(tr_v.at[b], [cvecs[cb], dvec])
                               + plsc.load_gather(dtab_v, [dvecs[cb], dvec]))
                        plsc.store_scatter(
                            ob_v.at[b], [dtv, zvec, drv, cvecs[cb]], acc)

        for b in range(NBUF):
            unpack_tok(b, b)
            gather_desc(b).start()

        @pl.loop(0, seq - NBUF, step=NBUF)
        def _(s0):
            for b in range(NBUF):
                s = s0 + b
                gather_desc(b).wait()
                unpack_tok(s + NBUF, b)

                @pl.when(s0 >= NBUF)
                def _():
                    write_desc(s - NBUF, b).wait()

                add_group(s, b)
                write_desc(s, b).start()
                gather_desc(b).start()

        for b in range(NBUF):
            s = seq - NBUF + b
            gather_desc(b).wait()
            write_desc(s - NBUF, b).wait()
            add_group(s, b)
            write_desc(s, b).start()
        for b in range(NBUF):
            s = seq - NBUF + b
            write_desc(s, b).wait()

    return kern


_kern = _build(4096, 200, 1000)


def kernel(token_inputs, diac_inputs, token_table, diac_table):
    B, S = token_inputs.shape
    packed = token_inputs * DPACK + diac_inputs
    pk = packed.T.reshape(S // 8, 8, NW, BT).transpose(0, 2, 1, 3)
    out = _kern(pk, token_table, diac_table)
    return out.transpose(2, 4, 0, 1, 3).reshape(B, S, D)


# speedup vs baseline: 1.1453x; 1.1453x over previous
import functools

import jax
import jax.numpy as jnp
from jax import lax
from jax.experimental import pallas as pl
from jax.experimental.pallas import tpu as pltpu
from jax.experimental.pallas import tpu_sc as plsc

D = 64
L = 16
NC = 2
NS = 16
NW = NC * NS
BT = 128
DT = D // 8
NBUF = 2
DPACK = 1024


def _build(batch, seq, diac_vocab):
    assert batch == NW * BT
    assert seq % 8 == 0
    st = seq // 8
    assert seq % NBUF == 0 and seq >= 2 * NBUF

    mesh = plsc.VectorSubcoreMesh(core_axis_name="c", subcore_axis_name="s")

    @functools.partial(
        pl.kernel,
        out_type=jax.ShapeDtypeStruct((seq, DT, NW, 8, BT), jnp.float32),
        mesh=mesh,
        scratch_types=[
            pltpu.VMEM((st, 1, 8, BT), jnp.int32),
            pltpu.VMEM((NBUF, BT), jnp.int32),
            pltpu.VMEM((diac_vocab, D), jnp.float32),
            pltpu.VMEM((NBUF, BT, D), jnp.float32),
            pltpu.VMEM((NBUF, DT, 1, 8, BT), jnp.float32),
            pltpu.SemaphoreType.DMA((NBUF,)),
            pltpu.SemaphoreType.DMA((NBUF,)),
        ],
        compiler_params=pltpu.CompilerParams(
            use_tc_tiling_on_sc=False, needs_layout_passes=False),
    )
    def kern(pk_idx_hbm, tok_tab_hbm, diac_tab_hbm, out_hbm,
             pk_v, ti_v, dtab_v, tr_v, ob_v, gsems, wsems):
        wid = lax.axis_index("s") * NC + lax.axis_index("c")

        pltpu.sync_copy(pk_idx_hbm.at[:, pl.ds(wid, 1)], pk_v)
        pltpu.sync_copy(diac_tab_hbm, dtab_v)

        iota = lax.iota(jnp.int32, L)
        zvec = jnp.zeros((L,), jnp.int32)
        cvecs = [iota + L * cb for cb in range(BT // L)]

        def pk_row(s):
            return pk_v.at[s // 8, 0, lax.rem(s, 8)]

        def unpack_tok(s, b):
            row = pk_row(s)
            for cb in range(BT // L):
                sl = pl.ds(cb * L, L)
                ti_v[b, sl] = lax.shift_right_logical(row[sl], 10)

        def gather_desc(b):
            return pltpu.make_async_copy(
                tok_tab_hbm.at[ti_v.at[b]], tr_v.at[b], gsems.at[b])

        def write_desc(s, b):
            return pltpu.make_async_copy(
                ob_v.at[b], out_hbm.at[s, pl.ds(0, DT), pl.ds(wid, 1)],
                wsems.at[b])

        def add_group(s, b):
            row = pk_row(s)
            dvecs = [row[pl.ds(cb * L, L)] & (DPACK - 1)
                     for cb in range(BT // L)]

            @plsc.parallel_loop(0, L, unroll=2)
            def _(j):
                pj = lax.rem(iota + j, jnp.int32(L))
                for db in range(D // L):
                    dvec = pj + L * db
                    dtv = dvec // 8
                    drv = lax.rem(dvec, 8)
                    for cb in range(BT // L):
                        acc = (plsc.load_gather(tr_v.at[b], [cvecs[cb], dvec])
                               + plsc.load_gather(dtab_v, [dvecs[cb], dvec]))
                        plsc.store_scatter(
                            ob_v.at[b], [dtv, zvec, drv, cvecs[cb]], acc)

        for b in range(NBUF):
            unpack_tok(b, b)
            gather_desc(b).start()

        @pl.loop(0, seq - NBUF, step=NBUF)
        def _(s0):
            for b in range(NBUF):
                s = s0 + b
                gather_desc(b).wait()
                unpack_tok(s + NBUF, b)

                @pl.when(s0 >= NBUF)
                def _():
                    write_desc(s - NBUF, b).wait()

                add_group(s, b)
                write_desc(s, b).start()
                gather_desc(b).start()

        for b in range(NBUF):
            s = seq - NBUF + b
            gather_desc(b).wait()
            write_desc(s - NBUF, b).wait()
            add_group(s, b)
            write_desc(s, b).start()
        for b in range(NBUF):
            s = seq - NBUF + b
            write_desc(s, b).wait()

    return kern


_kern = _build(4096, 200, 1000)


def kernel(token_inputs, diac_inputs, token_table, diac_table):
    B, S = token_inputs.shape
    def fmt(idx):
        return idx.T.reshape(S // 8, 8, NW, BT).transpose(0, 2, 1, 3)

    pk = fmt(token_inputs) * DPACK + fmt(diac_inputs)
    out = _kern(pk, token_table, diac_table)
    return out.transpose(2, 4, 0, 1, 3).reshape(B, S, D)


# speedup vs baseline: 1.4797x; 1.2920x over previous
import functools

import jax
import jax.numpy as jnp
from jax import lax
from jax.experimental import pallas as pl
from jax.experimental.pallas import tpu as pltpu
from jax.experimental.pallas import tpu_sc as plsc

D = 64
L = 16
NC = 2
NS = 16
NW = NC * NS
BT = 128
DT = D // 8
NBUF = 2


def _build(batch, seq, diac_vocab):
    assert batch == NW * BT
    assert seq % 8 == 0
    st = seq // 8
    assert seq % NBUF == 0 and seq >= 2 * NBUF

    mesh = plsc.VectorSubcoreMesh(core_axis_name="c", subcore_axis_name="s")

    @functools.partial(
        pl.kernel,
        out_type=jax.ShapeDtypeStruct((seq, DT, NW, 8, BT), jnp.float32),
        mesh=mesh,
        scratch_types=[
            pltpu.VMEM((st, 1, 8, BT), jnp.int32),
            pltpu.VMEM((NBUF, BT), jnp.int32),
            pltpu.VMEM((diac_vocab, D), jnp.float32),
            pltpu.VMEM((NBUF, BT, D), jnp.float32),
            pltpu.VMEM((NBUF, DT, 1, 8, BT), jnp.float32),
            pltpu.SemaphoreType.DMA((NBUF,)),
            pltpu.SemaphoreType.DMA((NBUF,)),
            pltpu.SemaphoreType.DMA((NBUF,)),
        ],
        compiler_params=pltpu.CompilerParams(
            use_tc_tiling_on_sc=False, needs_layout_passes=False),
    )
    def kern(tok_idx_hbm, diac_idx_hbm, tok_tab_hbm, diac_tab_hbm, out_hbm,
             di_v, ti_v, dtab_v, tr_v, ob_v, isems, gsems, wsems):
        wid = lax.axis_index("s") * NC + lax.axis_index("c")

        pltpu.sync_copy(diac_idx_hbm.at[:, pl.ds(wid, 1)], di_v)
        pltpu.sync_copy(diac_tab_hbm, dtab_v)

        iota = lax.iota(jnp.int32, L)
        zvec = jnp.zeros((L,), jnp.int32)
        cvecs = [iota + L * cb for cb in range(BT // L)]

        def idx_desc(s, b):
            return pltpu.make_async_copy(
                tok_idx_hbm.at[s // 8, wid, lax.rem(s, 8)],
                ti_v.at[b], isems.at[b])

        def gather_desc(b):
            return pltpu.make_async_copy(
                tok_tab_hbm.at[ti_v.at[b]], tr_v.at[b], gsems.at[b])

        def write_desc(s, b):
            return pltpu.make_async_copy(
                ob_v.at[b], out_hbm.at[s, pl.ds(0, DT), pl.ds(wid, 1)],
                wsems.at[b])

        def add_group(s, b):
            row = di_v.at[s // 8, 0, lax.rem(s, 8)]
            dvecs = [row[pl.ds(cb * L, L)] for cb in range(BT // L)]

            @plsc.parallel_loop(0, L, unroll=2)
            def _(j):
                pj = lax.rem(iota + j, jnp.int32(L))
                for db in range(D // L):
                    dvec = pj + L * db
                    dtv = dvec // 8
                    drv = lax.rem(dvec, 8)
                    for cb in range(BT // L):
                        acc = (plsc.load_gather(tr_v.at[b], [cvecs[cb], dvec])
                               + plsc.load_gather(dtab_v, [dvecs[cb], dvec]))
                        plsc.store_scatter(
                            ob_v.at[b], [dtv, zvec, drv, cvecs[cb]], acc)

        for b in range(NBUF):
            idx_desc(b, b).start()
        for b in range(NBUF):
            idx_desc(b, b).wait()
            gather_desc(b).start()

        @pl.loop(0, seq - NBUF, step=NBUF)
        def _(s0):
            for b in range(NBUF):
                s = s0 + b
                gather_desc(b).wait()
                idx_desc(s + NBUF, b).start()

                @pl.when(s0 >= NBUF)
                def _():
                    write_desc(s - NBUF, b).wait()

                add_group(s, b)
                write_desc(s, b).start()
                idx_desc(s + NBUF, b).wait()
                gather_desc(b).start()

        for b in range(NBUF):
            s = seq - NBUF + b
            gather_desc(b).wait()
            write_desc(s - NBUF, b).wait()
            add_group(s, b)
            write_desc(s, b).start()
        for b in range(NBUF):
            s = seq - NBUF + b
            write_desc(s, b).wait()

    return kern


_kern = _build(4096, 200, 1000)


def kernel(token_inputs, diac_inputs, token_table, diac_table):
    B, S = token_inputs.shape

    def fmt(idx):
        return idx.T.reshape(S // 8, 8, NW, BT).transpose(0, 2, 1, 3)

    out = _kern(fmt(token_inputs), fmt(diac_inputs), token_table, diac_table)
    return out.transpose(2, 4, 0, 1, 3).reshape(B, S, D)
